# baseline (speedup 1.0000x reference)
import functools

import jax
import jax.numpy as jnp
from jax import lax
from jax.experimental import pallas as pl
from jax.experimental.pallas import tpu as pltpu
from jax.experimental.pallas import tpu_sc as plsc

EMB_DIM = 32
LANES = 16
VOCAB_ROWS = 1000001
VPAD = 1000064
VTILES_FULL = VOCAB_ROWS // 128
VTAIL = VOCAB_ROWS - VTILES_FULL * 128
TROWS = VPAD // 4

_info = plsc.get_sparse_core_info()
_NC, _NS = _info.num_cores, _info.num_subcores
_NW = _NC * _NS


def _transpose_block(buf_in, buf_out, lane, n_valid_vq):
    for vq in range(n_valid_vq):
        for j in range(4):
            for eh in range(2):
                src = plsc.load_gather(
                    buf_in,
                    [lane + eh * 16, jnp.full((LANES,), vq * 4 + j, jnp.int32)])
                buf_out[vq, pl.ds(j * 32 + eh * 16, LANES)] = src


def _pack_body(embT, tail_lines, table, buf_in, buf_out, tail_buf, *, per_w):
    wid = lax.axis_index("s") * _NC + lax.axis_index("c")
    lane = lax.iota(jnp.int32, LANES)

    def tile_step(t, _):
        @pl.when(t < VTILES_FULL)
        def _():
            pltpu.sync_copy(embT.at[:, pl.ds(t * 128, 128)],
                            buf_in.at[:, pl.ds(0, 128)])
            _transpose_block(buf_in, buf_out, lane, 32)
            pltpu.sync_copy(buf_out, table.at[pl.ds(t * 32, 32), :])
        return 0

    lax.fori_loop(wid * per_w, (wid + 1) * per_w, tile_step, 0)

    @pl.when(wid == _NW - 1)
    def _():
        pltpu.sync_copy(tail_lines, tail_buf)
        pltpu.sync_copy(tail_buf, table.at[pl.ds(VTILES_FULL * 32, 24), :])


def _unit(xT, table, out, xbuf, rbuf, qbuf, lines, obuf, sem,
          lane, t_s, t_b, n_rows, s_dim, vocab_max):
    for k in range(8 * n_rows):
        v = xbuf[k // 8, pl.ds((k % 8) * 16, LANES)]
        v = jnp.minimum(jnp.maximum(v, 0), vocab_max)
        rbuf[pl.ds(k * 16, LANES)] = v >> 2
        qbuf[pl.ds(k * 16, LANES)] = (v & 3) * 32

    def row_step(r, _):
        s = t_s * 8 + r

        @pl.when(s < s_dim)
        def _():
            pltpu.async_copy(
                table.at[rbuf.at[pl.ds(r * 128, 128)]],
                lines.at[:, pl.ds(0, 128)], sem).wait()
            for c0 in range(8):
                rowi = lane + c0 * 16
                q = qbuf[pl.ds(r * 128 + c0 * 16, LANES)]
                for e in range(EMB_DIM):
                    vec = plsc.load_gather(lines, [rowi, q + e])
                    obuf[e, pl.ds(c0 * 16, LANES)] = vec
            pltpu.sync_copy(
                obuf, out.at[s, pl.ds(0, EMB_DIM), pl.ds(t_b * 128, 128)])
        return 0

    lax.fori_loop(0, n_rows, row_step, 0)


def _gather_body(xTp, table, out, xbuf, rbuf, qbuf, lines, obuf, sem,
                 *, n_ts, n_tb, s_dim, vocab_max):
    wid = lax.axis_index("s") * _NC + lax.axis_index("c")
    lane = lax.iota(jnp.int32, LANES)
    per_w = n_ts * n_tb // _NW

    def unit_step(u, _):
        t_s = u // n_tb
        t_b = u % n_tb
        pltpu.sync_copy(xTp.at[pl.ds(t_s * 8, 8), pl.ds(t_b * 128, 128)], xbuf)
        _unit(xTp, table, out, xbuf, rbuf, qbuf, lines, obuf, sem,
              lane, t_s, t_b, 8, s_dim, vocab_max)
        return 0

    lax.fori_loop(wid * per_w, (wid + 1) * per_w, unit_step, 0)


def kernel(x, emb):
    B, S = x.shape
    xT = jnp.transpose(x).astype(jnp.int32)
    embT = jnp.transpose(emb)

    s_pad = -(-S // 8) * 8
    xTp = jnp.pad(xT, ((0, s_pad - S), (0, 0)))

    tail = jnp.pad(emb[VTILES_FULL * 128:], ((0, 96 - VTAIL), (0, 0)))
    tail_lines = tail.reshape(24, 128)

    mesh = plsc.VectorSubcoreMesh(core_axis_name="c", subcore_axis_name="s")
    params = pltpu.CompilerParams(use_tc_tiling_on_sc=True,
                                  needs_layout_passes=False)

    per_w_pack = -(-VTILES_FULL // _NW)
    pack = pl.kernel(
        functools.partial(_pack_body, per_w=per_w_pack),
        out_type=jax.ShapeDtypeStruct((TROWS, 128), jnp.float32),
        mesh=mesh,
        scratch_types=[
            pltpu.VMEM((EMB_DIM, 129), jnp.float32),
            pltpu.VMEM((32, 128), jnp.float32),
            pltpu.VMEM((24, 128), jnp.float32),
        ],
        compiler_params=params,
    )
    table = pack(embT, tail_lines)

    gather = pl.kernel(
        functools.partial(_gather_body, n_ts=s_pad // 8, n_tb=B // 128,
                          s_dim=S, vocab_max=emb.shape[0] - 1),
        out_type=jax.ShapeDtypeStruct((S, EMB_DIM, B), jnp.float32),
        mesh=mesh,
        scratch_types=[
            pltpu.VMEM((8, 128), jnp.int32),
            pltpu.VMEM((1024,), jnp.int32),
            pltpu.VMEM((1024,), jnp.int32),
            pltpu.VMEM((128, 129), jnp.float32),
            pltpu.VMEM((EMB_DIM, 128), jnp.float32),
            pltpu.SemaphoreType.DMA,
        ],
        compiler_params=params,
    )
    out2 = gather(xTp, table)
    return jnp.transpose(out2, (2, 0, 1))


# speedup vs baseline: 1.2285x; 1.2285x over previous
import functools

import jax
import jax.numpy as jnp
from jax import lax
from jax.experimental import pallas as pl
from jax.experimental.pallas import tpu as pltpu
from jax.experimental.pallas import tpu_sc as plsc

EMB_DIM = 32
LANES = 16
VOCAB_ROWS = 1000001
VTILES_FULL = VOCAB_ROWS // 128
VTAIL = VOCAB_ROWS - VTILES_FULL * 128
TROWS = 250016

_info = plsc.get_sparse_core_info()
_NC, _NS = _info.num_cores, _info.num_subcores
_NW = _NC * _NS


def _transpose_block(buf_in, buf_out, lane):
    for vq in range(32):
        for j in range(4):
            for eh in range(2):
                src = plsc.load_gather(
                    buf_in,
                    [lane + eh * 16, jnp.full((LANES,), vq * 4 + j, jnp.int32)])
                buf_out[vq, pl.ds(j * 32 + eh * 16, LANES)] = src


def _pack_body(embT, tail_lines, table, in0, in1, buf_out, tail_buf,
               s_in0, s_in1, *, per_w):
    wid = lax.axis_index("s") * _NC + lax.axis_index("c")
    lane = lax.iota(jnp.int32, LANES)
    lo = wid * per_w
    bufs = [(in0, s_in0), (in1, s_in1)]

    def pair_step(i, _):
        t = lo + 2 * i
        for b in range(2):
            @pl.when(t + b < VTILES_FULL)
            def _(b=b):
                pltpu.async_copy(
                    embT.at[:, pl.ds((t + b) * 128, 128)],
                    bufs[b][0], bufs[b][1])
        for b in range(2):
            @pl.when(t + b < VTILES_FULL)
            def _(b=b):
                pltpu.make_async_copy(
                    embT.at[:, pl.ds((t + b) * 128, 128)],
                    bufs[b][0], bufs[b][1]).wait()
                _transpose_block(bufs[b][0], buf_out, lane)
                pltpu.sync_copy(buf_out, table.at[pl.ds((t + b) * 32, 32), :])
        return 0

    lax.fori_loop(0, per_w // 2, pair_step, 0)

    @pl.when(wid == _NW - 1)
    def _():
        pltpu.sync_copy(tail_lines, tail_buf)
        pltpu.sync_copy(tail_buf, table.at[pl.ds(VTILES_FULL * 32, 24), :])


def _extract(lines, obuf, qbuf, k, lane):
    for c0 in range(8):
        rowi = lane + c0 * 16
        q = qbuf[pl.ds(k * 128 + c0 * 16, LANES)]
        for e2 in range(EMB_DIM // 2):
            a = plsc.load_gather(lines, [rowi, q + (2 * e2)])
            b = plsc.load_gather(lines, [rowi, q + (2 * e2 + 1)])
            obuf[2 * e2, pl.ds(c0 * 16, LANES)] = a
            obuf[2 * e2 + 1, pl.ds(c0 * 16, LANES)] = b


def _gather_body(xTp, table, out, xbuf, rbuf, qbuf, lines0, lines1, obuf,
                 s_g0, s_g1, *, n_ts, n_tb, s_dim, vocab_max):
    wid = lax.axis_index("s") * _NC + lax.axis_index("c")
    lane = lax.iota(jnp.int32, LANES)
    per_w = n_ts * n_tb // _NW
    n_steps = per_w * 8
    u0 = wid * per_w
    lbufs = [(lines0, s_g0), (lines1, s_g1)]

    def pre_step(u, _):
        ug = u0 + u
        pltpu.sync_copy(
            xTp.at[pl.ds((ug // n_tb) * 8, 8),
                   pl.ds((ug % n_tb) * 128, 128)], xbuf)
        for k in range(64):
            v = xbuf[k // 8, pl.ds((k % 8) * 16, LANES)]
            v = jnp.minimum(jnp.maximum(v, 0), vocab_max)
            rbuf[pl.ds(u * 1024 + k * 16, LANES)] = v >> 2
            qbuf[pl.ds(u * 1024 + k * 16, LANES)] = (v & 3) * 32
        return 0

    lax.fori_loop(0, per_w, pre_step, 0)

    def s_of(k):
        ug = u0 + k // 8
        return (ug // n_tb) * 8 + (k % 8)

    def pair_step(i, _):
        k = 2 * i
        for b in range(2):
            @pl.when(s_of(k + b) < s_dim)
            def _(b=b):
                pltpu.async_copy(
                    table.at[rbuf.at[pl.ds((k + b) * 128, 128)]],
                    lbufs[b][0], lbufs[b][1])
        for b in range(2):
            @pl.when(s_of(k + b) < s_dim)
            def _(b=b):
                pltpu.make_async_copy(
                    table.at[rbuf.at[pl.ds((k + b) * 128, 128)]],
                    lbufs[b][0], lbufs[b][1]).wait()
                _extract(lbufs[b][0], obuf, qbuf, k + b, lane)
                ug = u0 + (k + b) // 8
                pltpu.sync_copy(
                    obuf, out.at[s_of(k + b), pl.ds(0, EMB_DIM),
                                 pl.ds((ug % n_tb) * 128, 128)])
        return 0

    lax.fori_loop(0, n_steps // 2, pair_step, 0)


def kernel(x, emb):
    B, S = x.shape
    xT = jnp.transpose(x).astype(jnp.int32)
    embT = jnp.transpose(emb)

    s_pad = -(-S // 8) * 8
    xTp = jnp.pad(xT, ((0, s_pad - S), (0, 0)))

    tail = jnp.pad(emb[VTILES_FULL * 128:], ((0, 96 - VTAIL), (0, 0)))
    tail_lines = tail.reshape(24, 128)

    mesh = plsc.VectorSubcoreMesh(core_axis_name="c", subcore_axis_name="s")
    params = pltpu.CompilerParams(use_tc_tiling_on_sc=True,
                                  needs_layout_passes=False)

    per_w_pack = 246
    pack = pl.kernel(
        functools.partial(_pack_body, per_w=per_w_pack),
        out_type=jax.ShapeDtypeStruct((TROWS, 128), jnp.float32),
        mesh=mesh,
        scratch_types=[
            pltpu.VMEM((EMB_DIM, 128), jnp.float32),
            pltpu.VMEM((EMB_DIM, 128), jnp.float32),
            pltpu.VMEM((32, 128), jnp.float32),
            pltpu.VMEM((24, 128), jnp.float32),
            pltpu.SemaphoreType.DMA,
            pltpu.SemaphoreType.DMA,
        ],
        compiler_params=params,
    )
    table = pack(embT, tail_lines)

    gather = pl.kernel(
        functools.partial(_gather_body, n_ts=s_pad // 8, n_tb=B // 128,
                          s_dim=S, vocab_max=emb.shape[0] - 1),
        out_type=jax.ShapeDtypeStruct((S, EMB_DIM, B), jnp.float32),
        mesh=mesh,
        scratch_types=[
            pltpu.VMEM((8, 128), jnp.int32),
            pltpu.VMEM((28 * 1024,), jnp.int32),
            pltpu.VMEM((28 * 1024,), jnp.int32),
            pltpu.VMEM((128, 128), jnp.float32),
            pltpu.VMEM((128, 128), jnp.float32),
            pltpu.VMEM((EMB_DIM, 128), jnp.float32),
            pltpu.SemaphoreType.DMA,
            pltpu.SemaphoreType.DMA,
        ],
        compiler_params=params,
    )
    out2 = gather(xTp, table)
    return jnp.transpose(out2, (2, 0, 1))


# speedup vs baseline: 1.2400x; 1.0094x over previous
import functools

import jax
import jax.numpy as jnp
from jax import lax
from jax.experimental import pallas as pl
from jax.experimental.pallas import tpu as pltpu
from jax.experimental.pallas import tpu_sc as plsc

EMB_DIM = 32
LANES = 16
VOCAB_ROWS = 1000001
VTILES_FULL = VOCAB_ROWS // 128
VTAIL = VOCAB_ROWS - VTILES_FULL * 128
TROWS = 250016

_info = plsc.get_sparse_core_info()
_NC, _NS = _info.num_cores, _info.num_subcores
_NW = _NC * _NS


def _transpose_block(buf_in, buf_out, lane):
    for vq in range(32):
        for j in range(4):
            for eh in range(2):
                src = plsc.load_gather(
                    buf_in,
                    [lane + eh * 16, jnp.full((LANES,), vq * 4 + j, jnp.int32)])
                buf_out[vq, pl.ds(j * 32 + eh * 16, LANES)] = src


def _pack_body(embT, tail_lines, table, in0, in1, buf_out, tail_buf,
               s_in0, s_in1, *, per_w):
    wid = lax.axis_index("s") * _NC + lax.axis_index("c")
    lane = lax.iota(jnp.int32, LANES)
    lo = wid * per_w
    bufs = [(in0, s_in0), (in1, s_in1)]

    def pair_step(i, _):
        t = lo + 2 * i
        for b in range(2):
            @pl.when(t + b < VTILES_FULL)
            def _(b=b):
                pltpu.async_copy(
                    embT.at[:, pl.ds((t + b) * 128, 128)],
                    bufs[b][0].at[:, pl.ds(0, 128)], bufs[b][1])
        for b in range(2):
            @pl.when(t + b < VTILES_FULL)
            def _(b=b):
                pltpu.make_async_copy(
                    embT.at[:, pl.ds((t + b) * 128, 128)],
                    bufs[b][0].at[:, pl.ds(0, 128)], bufs[b][1]).wait()
                _transpose_block(bufs[b][0], buf_out, lane)
                pltpu.sync_copy(buf_out, table.at[pl.ds((t + b) * 32, 32), :])
        return 0

    lax.fori_loop(0, per_w // 2, pair_step, 0)

    @pl.when(wid == _NW - 1)
    def _():
        pltpu.sync_copy(tail_lines, tail_buf)
        pltpu.sync_copy(tail_buf, table.at[pl.ds(VTILES_FULL * 32, 24), :])


def _extract(lines, obuf, qbuf, k, lane):
    for c0 in range(8):
        rowi = lane + c0 * 16
        q = qbuf[pl.ds(k * 128 + c0 * 16, LANES)]
        for e2 in range(EMB_DIM // 2):
            a = plsc.load_gather(lines, [rowi, q + (2 * e2)])
            b = plsc.load_gather(lines, [rowi, q + (2 * e2 + 1)])
            obuf[2 * e2, pl.ds(c0 * 16, LANES)] = a
            obuf[2 * e2 + 1, pl.ds(c0 * 16, LANES)] = b


def _gather_body(xTp, table, out, xbuf, rbuf, qbuf, lines0, lines1, obuf,
                 s_g0, s_g1, *, n_ts, n_tb, s_dim, vocab_max):
    wid = lax.axis_index("s") * _NC + lax.axis_index("c")
    lane = lax.iota(jnp.int32, LANES)
    per_w = n_ts * n_tb // _NW
    n_steps = per_w * 8
    u0 = wid * per_w
    lbufs = [(lines0, s_g0), (lines1, s_g1)]

    def pre_step(u, _):
        ug = u0 + u
        pltpu.sync_copy(
            xTp.at[pl.ds((ug // n_tb) * 8, 8),
                   pl.ds((ug % n_tb) * 128, 128)], xbuf)
        for k in range(64):
            v = xbuf[k // 8, pl.ds((k % 8) * 16, LANES)]
            v = jnp.minimum(jnp.maximum(v, 0), vocab_max)
            rbuf[pl.ds(u * 1024 + k * 16, LANES)] = v >> 2
            qbuf[pl.ds(u * 1024 + k * 16, LANES)] = (v & 3) * 32
        return 0

    lax.fori_loop(0, per_w, pre_step, 0)

    def s_of(k):
        ug = u0 + k // 8
        return (ug // n_tb) * 8 + (k % 8)

    def pair_step(i, _):
        k = 2 * i
        for b in range(2):
            @pl.when(s_of(k + b) < s_dim)
            def _(b=b):
                pltpu.async_copy(
                    table.at[rbuf.at[pl.ds((k + b) * 128, 128)]],
                    lbufs[b][0], lbufs[b][1])
        for b in range(2):
            @pl.when(s_of(k + b) < s_dim)
            def _(b=b):
                pltpu.make_async_copy(
                    table.at[rbuf.at[pl.ds((k + b) * 128, 128)]],
                    lbufs[b][0], lbufs[b][1]).wait()
                _extract(lbufs[b][0], obuf, qbuf, k + b, lane)
                ug = u0 + (k + b) // 8
                pltpu.sync_copy(
                    obuf, out.at[s_of(k + b), pl.ds(0, EMB_DIM),
                                 pl.ds((ug % n_tb) * 128, 128)])
        return 0

    lax.fori_loop(0, n_steps // 2, pair_step, 0)


def kernel(x, emb):
    B, S = x.shape
    xT = jnp.transpose(x).astype(jnp.int32)
    embT = jnp.transpose(emb)

    s_pad = -(-S // 8) * 8
    xTp = jnp.pad(xT, ((0, s_pad - S), (0, 0)))

    tail = jnp.pad(emb[VTILES_FULL * 128:], ((0, 96 - VTAIL), (0, 0)))
    tail_lines = tail.reshape(24, 128)

    mesh = plsc.VectorSubcoreMesh(core_axis_name="c", subcore_axis_name="s")
    params = pltpu.CompilerParams(use_tc_tiling_on_sc=True,
                                  needs_layout_passes=False)

    per_w_pack = 246
    pack = pl.kernel(
        functools.partial(_pack_body, per_w=per_w_pack),
        out_type=jax.ShapeDtypeStruct((TROWS, 128), jnp.float32),
        mesh=mesh,
        scratch_types=[
            pltpu.VMEM((EMB_DIM, 129), jnp.float32),
            pltpu.VMEM((EMB_DIM, 129), jnp.float32),
            pltpu.VMEM((32, 128), jnp.float32),
            pltpu.VMEM((24, 128), jnp.float32),
            pltpu.SemaphoreType.DMA,
            pltpu.SemaphoreType.DMA,
        ],
        compiler_params=params,
    )
    table = pack(embT, tail_lines)

    gather = pl.kernel(
        functools.partial(_gather_body, n_ts=s_pad // 8, n_tb=B // 128,
                          s_dim=S, vocab_max=emb.shape[0] - 1),
        out_type=jax.ShapeDtypeStruct((S, EMB_DIM, B), jnp.float32),
        mesh=mesh,
        scratch_types=[
            pltpu.VMEM((8, 128), jnp.int32),
            pltpu.VMEM((28 * 1024,), jnp.int32),
            pltpu.VMEM((28 * 1024,), jnp.int32),
            pltpu.VMEM((128, 128), jnp.float32),
            pltpu.VMEM((128, 128), jnp.float32),
            pltpu.VMEM((EMB_DIM, 128), jnp.float32),
            pltpu.SemaphoreType.DMA,
            pltpu.SemaphoreType.DMA,
        ],
        compiler_params=params,
    )
    out2 = gather(xTp, table)
    return jnp.transpose(out2, (2, 0, 1))


# speedup vs baseline: 1.7784x; 1.4342x over previous
import functools

import jax
import jax.numpy as jnp
from jax import lax
from jax.experimental import pallas as pl
from jax.experimental.pallas import tpu as pltpu
from jax.experimental.pallas import tpu_sc as plsc

EMB_DIM = 32
LANES = 16

_info = plsc.get_sparse_core_info()
_NC, _NS = _info.num_cores, _info.num_subcores
_NW = _NC * _NS


def _extract(lines, obuf, lane):
    for c0 in range(8):
        rowi = lane + c0 * 16
        for e2 in range(EMB_DIM // 2):
            a = plsc.load_gather(
                lines, [rowi, jnp.full((LANES,), 2 * e2, jnp.int32)])
            b = plsc.load_gather(
                lines, [rowi, jnp.full((LANES,), 2 * e2 + 1, jnp.int32)])
            obuf[2 * e2, pl.ds(c0 * 16, LANES)] = a
            obuf[2 * e2 + 1, pl.ds(c0 * 16, LANES)] = b


def _gather_body(xTp, table, out, xbuf, rbuf, lines0, lines1, obuf,
                 s_g0, s_g1, *, n_ts, n_tb, s_dim, vocab_max):
    wid = lax.axis_index("s") * _NC + lax.axis_index("c")
    lane = lax.iota(jnp.int32, LANES)
    per_w = n_ts * n_tb // _NW
    n_steps = per_w * 8
    u0 = wid * per_w
    lbufs = [(lines0, s_g0), (lines1, s_g1)]

    def pre_step(u, _):
        ug = u0 + u
        pltpu.sync_copy(
            xTp.at[pl.ds((ug // n_tb) * 8, 8),
                   pl.ds((ug % n_tb) * 128, 128)], xbuf)
        for k in range(64):
            v = xbuf[k // 8, pl.ds((k % 8) * 16, LANES)]
            rbuf[pl.ds(u * 1024 + k * 16, LANES)] = (
                jnp.minimum(jnp.maximum(v, 0), vocab_max))
        return 0

    lax.fori_loop(0, per_w, pre_step, 0)

    def s_of(k):
        ug = u0 + k // 8
        return (ug // n_tb) * 8 + (k % 8)

    def pair_step(i, _):
        k = 2 * i
        for b in range(2):
            @pl.when(s_of(k + b) < s_dim)
            def _(b=b):
                pltpu.async_copy(
                    table.at[rbuf.at[pl.ds((k + b) * 128, 128)]],
                    lbufs[b][0], lbufs[b][1])
        for b in range(2):
            @pl.when(s_of(k + b) < s_dim)
            def _(b=b):
                pltpu.make_async_copy(
                    table.at[rbuf.at[pl.ds((k + b) * 128, 128)]],
                    lbufs[b][0], lbufs[b][1]).wait()
                _extract(lbufs[b][0], obuf, lane)
                ug = u0 + (k + b) // 8
                pltpu.sync_copy(
                    obuf, out.at[s_of(k + b), pl.ds(0, EMB_DIM),
                                 pl.ds((ug % n_tb) * 128, 128)])
        return 0

    lax.fori_loop(0, n_steps // 2, pair_step, 0)


def kernel(x, emb):
    B, S = x.shape
    V = emb.shape[0]
    xT = jnp.transpose(x).astype(jnp.int32)

    s_pad = -(-S // 8) * 8
    xTp = jnp.pad(xT, ((0, s_pad - S), (0, 0)))

    v_pad = -(-V // 8) * 8
    table = jnp.pad(emb, ((0, v_pad - V), (0, 128 - EMB_DIM)))

    mesh = plsc.VectorSubcoreMesh(core_axis_name="c", subcore_axis_name="s")
    params = pltpu.CompilerParams(use_tc_tiling_on_sc=True,
                                  needs_layout_passes=False)

    gather = pl.kernel(
        functools.partial(_gather_body, n_ts=s_pad // 8, n_tb=B // 128,
                          s_dim=S, vocab_max=V - 1),
        out_type=jax.ShapeDtypeStruct((S, EMB_DIM, B), jnp.float32),
        mesh=mesh,
        scratch_types=[
            pltpu.VMEM((8, 128), jnp.int32),
            pltpu.VMEM((28 * 1024,), jnp.int32),
            pltpu.VMEM((128, 128), jnp.float32),
            pltpu.VMEM((128, 128), jnp.float32),
            pltpu.VMEM((EMB_DIM, 128), jnp.float32),
            pltpu.SemaphoreType.DMA,
            pltpu.SemaphoreType.DMA,
        ],
        compiler_params=params,
    )
    out2 = gather(xTp, table)
    return jnp.transpose(out2, (2, 0, 1))
